# Initial kernel scaffold; baseline (speedup 1.0000x reference)
import jax
import jax.numpy as jnp
from jax.experimental import pallas as pl


def kernel(predicted, target):
    raise NotImplementedError("write your pallas kernel here")


# speedup vs baseline: 2.7970x; 2.7970x over previous
import functools

import jax
import jax.numpy as jnp
from jax import lax
from jax.experimental import pallas as pl
from jax.experimental.pallas import tpu as pltpu
from jax.experimental.pallas import tpu_sc as plsc

N = 8388608
NC = 2
NS = 16
L = 16
NW = NC * NS
PER_W = N // NW
CHUNK = 16384
NCH = PER_W // CHUNK


def _wsum_chunk(p_ref, t_ref, acc):
    def body(j, acc):
        p = p_ref[pl.ds(j * L, L)]
        t = t_ref[pl.ds(j * L, L)]
        w = jnp.where(
            t > 10.0, 0.0,
            jnp.where(
                t > 1.0, 1.0,
                jnp.where(
                    t > 0.5, 2.0,
                    jnp.where(
                        t > -0.5, 5.0,
                        jnp.where(t > -1.0, 2.0,
                                  jnp.where(t > -10.0, 1.0, 0.0))))))
        d = p - t
        return acc + w * (d * d)

    return lax.fori_loop(0, CHUNK // L, body, acc)


def _sc_body(pred_hbm, targ_hbm, out_hbm, p0, t0, p1, t1, accv, sem0, sem1):
    c = lax.axis_index("c")
    s = lax.axis_index("s")
    wid = s * NC + c
    base = wid * PER_W

    bufs = ((p0, t0, sem0), (p1, t1, sem1))

    def start(k):
        pb, tb, sem = bufs[k % 2]
        off = base + k * CHUNK
        cp = pltpu.async_copy(pred_hbm.at[pl.ds(off, CHUNK)], pb, sem)
        ct = pltpu.async_copy(targ_hbm.at[pl.ds(off, CHUNK)], tb, sem)
        return cp, ct

    acc = jnp.zeros((L,), jnp.float32)
    inflight = {0: start(0)}
    for k in range(NCH):
        if k + 1 < NCH:
            inflight[k + 1] = start(k + 1)
        cp, ct = inflight.pop(k)
        cp.wait()
        ct.wait()
        pb, tb, _ = bufs[k % 2]
        acc = _wsum_chunk(pb, tb, acc)

    accv[...] = acc
    pltpu.sync_copy(accv, out_hbm.at[pl.ds(wid * L, L)])


_sc_call = functools.partial(
    pl.kernel,
    mesh=plsc.VectorSubcoreMesh(core_axis_name="c", subcore_axis_name="s"),
    out_type=jax.ShapeDtypeStruct((NW * L,), jnp.float32),
    scratch_types=[
        pltpu.VMEM((CHUNK,), jnp.float32),
        pltpu.VMEM((CHUNK,), jnp.float32),
        pltpu.VMEM((CHUNK,), jnp.float32),
        pltpu.VMEM((CHUNK,), jnp.float32),
        pltpu.VMEM((L,), jnp.float32),
        pltpu.SemaphoreType.DMA,
        pltpu.SemaphoreType.DMA,
    ],
)(_sc_body)


def kernel(predicted, target):
    partials = _sc_call(predicted, target)
    return jnp.sum(partials) * (1.0 / 16.0)


# speedup vs baseline: 3.6823x; 1.3165x over previous
import functools

import jax
import jax.numpy as jnp
from jax import lax
from jax.experimental import pallas as pl
from jax.experimental.pallas import tpu as pltpu
from jax.experimental.pallas import tpu_sc as plsc

N = 8388608
NC = 2
NS = 16
L = 16
NW = NC * NS
PER_W = N // NW
CHUNK = 16384
NCH = PER_W // CHUNK


UNROLL = 4


def _wd2(p, t):
    a = jnp.abs(t)
    w = jnp.where(a > 10.0, 0.0,
                  jnp.where(a > 1.0, 1.0,
                            jnp.where(a > 0.5, 2.0, 5.0)))
    d = p - t
    return w * (d * d)


def _wsum_chunk(p_ref, t_ref, accs):
    def body(j, accs):
        base = j * (L * UNROLL)
        return tuple(
            accs[u] + _wd2(p_ref[pl.ds(base + u * L, L)],
                           t_ref[pl.ds(base + u * L, L)])
            for u in range(UNROLL))

    return lax.fori_loop(0, CHUNK // (L * UNROLL), body, accs)


def _sc_body(pred_hbm, targ_hbm, out_hbm, p0, t0, p1, t1, accv, sem0, sem1):
    c = lax.axis_index("c")
    s = lax.axis_index("s")
    wid = s * NC + c
    base = wid * PER_W

    bufs = ((p0, t0, sem0), (p1, t1, sem1))

    def start(k):
        pb, tb, sem = bufs[k % 2]
        off = base + k * CHUNK
        cp = pltpu.async_copy(pred_hbm.at[pl.ds(off, CHUNK)], pb, sem)
        ct = pltpu.async_copy(targ_hbm.at[pl.ds(off, CHUNK)], tb, sem)
        return cp, ct

    accs = tuple(jnp.zeros((L,), jnp.float32) for _ in range(UNROLL))
    inflight = {0: start(0)}
    for k in range(NCH):
        if k + 1 < NCH:
            inflight[k + 1] = start(k + 1)
        cp, ct = inflight.pop(k)
        cp.wait()
        ct.wait()
        pb, tb, _ = bufs[k % 2]
        accs = _wsum_chunk(pb, tb, accs)

    accv[...] = (accs[0] + accs[1]) + (accs[2] + accs[3])
    pltpu.sync_copy(accv, out_hbm.at[pl.ds(wid * L, L)])


_sc_call = functools.partial(
    pl.kernel,
    mesh=plsc.VectorSubcoreMesh(core_axis_name="c", subcore_axis_name="s"),
    out_type=jax.ShapeDtypeStruct((NW * L,), jnp.float32),
    scratch_types=[
        pltpu.VMEM((CHUNK,), jnp.float32),
        pltpu.VMEM((CHUNK,), jnp.float32),
        pltpu.VMEM((CHUNK,), jnp.float32),
        pltpu.VMEM((CHUNK,), jnp.float32),
        pltpu.VMEM((L,), jnp.float32),
        pltpu.SemaphoreType.DMA,
        pltpu.SemaphoreType.DMA,
    ],
)(_sc_body)


def kernel(predicted, target):
    partials = _sc_call(predicted, target)
    return jnp.sum(partials) * (1.0 / 16.0)
